# baseline (speedup 1.0000x reference)
import functools

import jax
import jax.numpy as jnp
import numpy as np
from jax import lax
from jax.experimental import pallas as pl
from jax.experimental.pallas import tpu as pltpu
from jax.experimental.pallas import tpu_sc as plsc

N = 8192
C = 8
H = 256
NHID = 6
B = 512
NB = N // B + C
NP = NB * B
RCH = 4
NPEW = 128

_NC = 2
_NS = 16
_NW = _NC * _NS
_PW = N // _NW
_IC = _PW // 128


def _pe(v, degree):
    s = jnp.sin((2.0 * np.pi) * v)
    c = jnp.cos((2.0 * np.pi) * v)
    outs = [v, s, c]
    for _ in range(degree - 1):
        s, c = 2.0 * s * c, 1.0 - 2.0 * s * s
        outs.append(s)
        outs.append(c)
    return jnp.concatenate(outs, axis=-1)


def _mlp(h, w_in, b_in, w_h_ref, b_h_ref, w_out, b_out, pre=lambda a, b: a[b]):
    h = jax.nn.relu(jnp.dot(h, w_in, preferred_element_type=jnp.float32) + b_in)
    for i in range(NHID):
        h = jax.nn.relu(
            jnp.dot(h, pre(w_h_ref, i), preferred_element_type=jnp.float32)
            + pre(b_h_ref, i))
    return jnp.dot(h, w_out, preferred_element_type=jnp.float32) + b_out


def _router_body(x_ref, proc_ref, wi_ref, bi_ref, wh_ref, bh_ref, wo_ref,
                 bo_ref, probs_ref, x29_ref, ps_ref, eid_ref):
    rows = N // RCH
    proc = jnp.broadcast_to(proc_ref[...], (rows, 2))
    wi = wi_ref[...]
    bi = bi_ref[...]
    wo = wo_ref[...]
    bo = bo_ref[...]
    for c in range(RCH):
        x = x_ref[c * rows:(c + 1) * rows, :]
        pe4 = _pe(x, 4)
        x29_ref[c * rows:(c + 1) * rows, 0:27] = pe4
        x29_ref[c * rows:(c + 1) * rows, 27:29] = proc
        h = jnp.concatenate([pe4[:, 0:9], proc], axis=-1)
        logits = _mlp(h, wi, bi, wh_ref, bh_ref, wo, bo)
        m = jnp.max(logits, axis=1, keepdims=True)
        e = jnp.exp(logits - m)
        probs_ref[c * rows:(c + 1) * rows, :] = e / jnp.sum(e, axis=1,
                                                            keepdims=True)
    p = probs_ref[...]
    pm = jnp.max(p, axis=1, keepdims=True)
    col = lax.broadcasted_iota(jnp.int32, (N, C), 1)
    idx = jnp.min(jnp.where(p == pm, col, C), axis=1)
    onehot = (col == idx[:, None]).astype(jnp.int32)
    cs = onehot
    s = 1
    while s < N:
        shifted = jnp.concatenate(
            [jnp.zeros((s, C), jnp.int32), cs[:N - s, :]], axis=0)
        cs = cs + shifted
        s *= 2
    rank = jnp.sum(cs * onehot, axis=1) - 1
    counts = cs[N - 1:N, :]
    pg = ((counts + (B - 1)) // B) * B
    poff_incl = pg
    s = 1
    while s < C:
        poff_incl = poff_incl + jnp.concatenate(
            [jnp.zeros((1, s), jnp.int32), poff_incl[:, :C - s]], axis=1)
        s *= 2
    poff_excl = poff_incl - pg
    ps_ref[...] = jnp.sum(onehot * poff_excl, axis=1) + rank
    bstart = lax.broadcasted_iota(jnp.int32, (NB, C), 0) * B
    ge = (bstart >= jnp.broadcast_to(poff_incl, (NB, C))).astype(jnp.int32)
    eid_ref[...] = jnp.minimum(jnp.sum(ge, axis=1), C - 1)


def _router_call(x, proc, wi, bi, wh, bh, wo, bo):
    return pl.pallas_call(
        _router_body,
        out_shape=(jax.ShapeDtypeStruct((N, C), jnp.float32),
                   jax.ShapeDtypeStruct((N, NPEW), jnp.float32),
                   jax.ShapeDtypeStruct((N,), jnp.int32),
                   jax.ShapeDtypeStruct((NB,), jnp.int32)),
    )(x, proc, wi, bi, wh, bh, wo, bo)


@functools.lru_cache(maxsize=None)
def _sc_kernels():
    mesh = plsc.VectorSubcoreMesh(core_axis_name="c", subcore_axis_name="s")
    scratch = [
        pltpu.VMEM((_IC, 128), jnp.int32),
        pltpu.VMEM((_PW, NPEW), jnp.float32),
        pltpu.SemaphoreType.DMA,
    ]

    @functools.partial(
        pl.kernel,
        out_type=jax.ShapeDtypeStruct((NP, NPEW), jnp.float32),
        mesh=mesh, scratch_types=scratch)
    def sc_scatter(x16_hbm, ps2_hbm, out_hbm, idx_v, rows_v, sem):
        wid = lax.axis_index("s") * _NC + lax.axis_index("c")
        base = wid * _PW
        pltpu.sync_copy(ps2_hbm.at[pl.ds(wid * _IC, _IC)], idx_v)
        pltpu.sync_copy(x16_hbm.at[pl.ds(base, _PW)], rows_v)
        for cidx in range(_IC):
            pltpu.async_copy(rows_v.at[pl.ds(cidx * 128, 128)],
                             out_hbm.at[idx_v.at[cidx]], sem).wait()

    @functools.partial(
        pl.kernel,
        out_type=jax.ShapeDtypeStruct((N, NPEW), jnp.float32),
        mesh=mesh, scratch_types=scratch)
    def sc_gather(ys_hbm, ps2_hbm, out_hbm, idx_v, rows_v, sem):
        wid = lax.axis_index("s") * _NC + lax.axis_index("c")
        base = wid * _PW
        pltpu.sync_copy(ps2_hbm.at[pl.ds(wid * _IC, _IC)], idx_v)
        for cidx in range(_IC):
            pltpu.async_copy(ys_hbm.at[idx_v.at[cidx]],
                             rows_v.at[pl.ds(cidx * 128, 128)], sem).wait()
        pltpu.sync_copy(rows_v, out_hbm.at[pl.ds(base, _PW)])

    return sc_scatter, sc_gather


def _sc_scatter_call(x16, ps2):
    return _sc_kernels()[0](x16, ps2)


def _sc_gather_call(ys, ps2):
    return _sc_kernels()[1](ys, ps2)


def _expert_body(eid_ref, xs_ref, proc_ref, twi, tbi, twh, tbh, two, tbo,
                 swi, sbi, swh, sbh, swo, sbo, out_ref):
    del eid_ref
    ht = xs_ref[...][:, 0:29]
    proc = jnp.broadcast_to(proc_ref[...], (B, 2))
    pre3 = lambda r, i: r[0, i]
    uv = jax.nn.sigmoid(_mlp(ht, twi[0], tbi[0, 0], twh, tbh, two[0],
                             tbo[0, 0], pre=pre3))
    hs = jnp.concatenate([_pe(uv, 4), proc], axis=-1)
    rec = _mlp(hs, swi[0], sbi[0, 0], swh, sbh, swo[0], sbo[0, 0], pre=pre3)
    out_ref[:, 0:2] = uv
    out_ref[:, 2:5] = rec


def _expert_call(eid, xs, proc, twi, tbi, twh, tbh, two, tbo,
                 swi, sbi, swh, sbh, swo, sbo):
    tbi, tbo, sbi, sbo = (a.reshape(C, 1, -1) for a in (tbi, tbo, sbi, sbo))
    e3 = lambda b, eid_ref: (eid_ref[b], 0, 0)
    e4 = lambda b, eid_ref: (eid_ref[b], 0, 0, 0)
    grid_spec = pltpu.PrefetchScalarGridSpec(
        num_scalar_prefetch=1,
        grid=(NB,),
        in_specs=[
            pl.BlockSpec((B, NPEW), lambda b, eid_ref: (b, 0)),
            pl.BlockSpec((1, 2), lambda b, eid_ref: (0, 0)),
            pl.BlockSpec((1, 29, H), e3),
            pl.BlockSpec((1, 1, H), e3),
            pl.BlockSpec((1, NHID, H, H), e4),
            pl.BlockSpec((1, NHID, H), e3),
            pl.BlockSpec((1, H, 2), e3),
            pl.BlockSpec((1, 1, 2), e3),
            pl.BlockSpec((1, 20, H), e3),
            pl.BlockSpec((1, 1, H), e3),
            pl.BlockSpec((1, NHID, H, H), e4),
            pl.BlockSpec((1, NHID, H), e3),
            pl.BlockSpec((1, H, 3), e3),
            pl.BlockSpec((1, 1, 3), e3),
        ],
        out_specs=pl.BlockSpec((B, NPEW), lambda b, eid_ref: (b, 0)),
    )
    return pl.pallas_call(
        _expert_body,
        grid_spec=grid_spec,
        out_shape=jax.ShapeDtypeStruct((NP, NPEW), jnp.float32),
    )(eid, xs, proc, twi, tbi, twh, tbh, two, tbo,
      swi, sbi, swh, sbh, swo, sbo)


def kernel(x, proc_params, cW_in, cb_in, cW_h, cb_h, cW_out, cb_out,
           tW_in, tb_in, tW_h, tb_h, tW_out, tb_out,
           sW_in, sb_in, sW_h, sb_h, sW_out, sb_out):
    probs, x29, ps, eid = _router_call(x, proc_params, cW_in, cb_in,
                                       cW_h, cb_h, cW_out, cb_out)
    ps2 = ps.reshape(N // 128, 128)
    xs = _sc_scatter_call(x29, ps2)
    ys = _expert_call(eid, xs, proc_params, tW_in, tb_in, tW_h, tb_h,
                      tW_out, tb_out, sW_in, sb_in, sW_h, sb_h,
                      sW_out, sb_out)
    g = _sc_gather_call(ys, ps2)
    return jnp.concatenate([probs, g[:, 0:5]], axis=-1)


# speedup vs baseline: 1.1400x; 1.1400x over previous
import functools

import jax
import jax.numpy as jnp
import numpy as np
from jax import lax
from jax.experimental import pallas as pl
from jax.experimental.pallas import tpu as pltpu
from jax.experimental.pallas import tpu_sc as plsc

N = 8192
C = 8
H = 256
NHID = 6
B = 512
NB = N // B + C
NP = NB * B
LCH = 4
NPEW = 128

_NC = 2
_NS = 16
_NW = _NC * _NS
_PW = N // _NW
_IC = _PW // 128


def _dgt(w, a):
    return lax.dot_general(w, a, (((0,), (0,)), ((), ())),
                           preferred_element_type=jnp.float32)


def _pe_t(v, degree):
    s = jnp.sin((2.0 * np.pi) * v)
    c = jnp.cos((2.0 * np.pi) * v)
    outs = [v, s, c]
    for _ in range(degree - 1):
        s, c = 2.0 * s * c, 1.0 - 2.0 * s * s
        outs.append(s)
        outs.append(c)
    return outs


def _mlp_t(h, w_in, b_in, w_h_ref, b_h_ref, w_out, b_out,
           pre=lambda a, b: a[b]):
    h = jax.nn.relu(_dgt(w_in, h) + b_in[:, None])
    for i in range(NHID):
        h = jax.nn.relu(_dgt(pre(w_h_ref, i), h) + pre(b_h_ref, i)[:, None])
    return _dgt(w_out, h) + b_out[:, None]


def _router_body(xt_ref, proc_ref, wi_ref, bi_ref, wh_ref, bh_ref, wo_ref,
                 bo_ref, probst_ref, ps_ref, eid_ref):
    nn = N // LCH
    proc = jnp.broadcast_to(jnp.transpose(proc_ref[...]), (2, nn))
    wi = wi_ref[...]
    bi = bi_ref[...]
    wo = wo_ref[...]
    bo = bo_ref[...]
    for ch in range(LCH):
        sl = pl.ds(ch * nn, nn)
        xc = xt_ref[:, sl]
        pe1 = _pe_t(xc, 1)
        hc = jnp.concatenate(pe1 + [proc], axis=0)
        logits = _mlp_t(hc, wi, bi, wh_ref, bh_ref, wo, bo)
        m = jnp.max(logits, axis=0, keepdims=True)
        e = jnp.exp(logits - m)
        probst_ref[:, sl] = e / jnp.sum(e, axis=0, keepdims=True)
    p = probst_ref[...]
    pm = jnp.max(p, axis=0, keepdims=True)
    row = lax.broadcasted_iota(jnp.int32, (C, N), 0)
    idx = jnp.min(jnp.where(p == pm, row, C), axis=0, keepdims=True)
    onehot = (row == idx).astype(jnp.int32)
    cs = onehot
    s = 1
    while s < N:
        shifted = jnp.concatenate(
            [jnp.zeros((C, s), jnp.int32), cs[:, :N - s]], axis=1)
        cs = cs + shifted
        s *= 2
    rank = jnp.sum(cs * onehot, axis=0, keepdims=True) - 1
    counts = cs[:, N - 1:N]
    pg = ((counts + (B - 1)) // B) * B
    poff_incl = pg
    s = 1
    while s < C:
        poff_incl = poff_incl + jnp.concatenate(
            [jnp.zeros((s, 1), jnp.int32), poff_incl[:C - s, :]], axis=0)
        s *= 2
    poff_excl = poff_incl - pg
    ps_ref[...] = jnp.sum(onehot * poff_excl, axis=0, keepdims=True) + rank
    bstart = lax.broadcasted_iota(jnp.int32, (C, NB), 1) * B
    ge = (bstart >= poff_incl).astype(jnp.int32)
    eid_ref[...] = jnp.minimum(jnp.sum(ge, axis=0), C - 1)


def _router_call(xt, proc, wi, bi, wh, bh, wo, bo):
    return pl.pallas_call(
        _router_body,
        out_shape=(jax.ShapeDtypeStruct((C, N), jnp.float32),
                   jax.ShapeDtypeStruct((1, N), jnp.int32),
                   jax.ShapeDtypeStruct((NB,), jnp.int32)),
    )(xt, proc, wi, bi, wh, bh, wo, bo)


@functools.lru_cache(maxsize=None)
def _sc_kernels():
    mesh = plsc.VectorSubcoreMesh(core_axis_name="c", subcore_axis_name="s")
    scratch = [
        pltpu.VMEM((_IC, 128), jnp.int32),
        pltpu.VMEM((_PW, NPEW), jnp.float32),
        pltpu.SemaphoreType.DMA,
    ]

    @functools.partial(
        pl.kernel,
        out_type=jax.ShapeDtypeStruct((NP, NPEW), jnp.float32),
        mesh=mesh, scratch_types=scratch)
    def sc_scatter(x16_hbm, ps2_hbm, out_hbm, idx_v, rows_v, sem):
        wid = lax.axis_index("s") * _NC + lax.axis_index("c")
        base = wid * _PW
        pltpu.sync_copy(ps2_hbm.at[pl.ds(wid * _IC, _IC)], idx_v)
        pltpu.sync_copy(x16_hbm.at[pl.ds(base, _PW)], rows_v)
        for cidx in range(_IC):
            pltpu.async_copy(rows_v.at[pl.ds(cidx * 128, 128)],
                             out_hbm.at[idx_v.at[cidx]], sem).wait()

    @functools.partial(
        pl.kernel,
        out_type=jax.ShapeDtypeStruct((N, NPEW), jnp.float32),
        mesh=mesh, scratch_types=scratch)
    def sc_gather(ys_hbm, ps2_hbm, out_hbm, idx_v, rows_v, sem):
        wid = lax.axis_index("s") * _NC + lax.axis_index("c")
        base = wid * _PW
        pltpu.sync_copy(ps2_hbm.at[pl.ds(wid * _IC, _IC)], idx_v)
        for cidx in range(_IC):
            pltpu.async_copy(ys_hbm.at[idx_v.at[cidx]],
                             rows_v.at[pl.ds(cidx * 128, 128)], sem).wait()
        pltpu.sync_copy(rows_v, out_hbm.at[pl.ds(base, _PW)])

    return sc_scatter, sc_gather


def _sc_scatter_call(x16, ps2):
    return _sc_kernels()[0](x16, ps2)


def _sc_gather_call(ys, ps2):
    return _sc_kernels()[1](ys, ps2)


def _expert_body(eid_ref, xs_ref, proc_ref, twi, tbi, twh, tbh, two, tbo,
                 swi, sbi, swh, sbh, swo, sbo, out_ref):
    del eid_ref
    xbt = jnp.transpose(xs_ref[...][:, 0:3])
    proc = jnp.broadcast_to(jnp.transpose(proc_ref[...]), (2, B))
    ht = jnp.concatenate(_pe_t(xbt, 4) + [proc], axis=0)
    pre3 = lambda r, i: r[0, i]
    uvt = jax.nn.sigmoid(_mlp_t(ht, twi[0], tbi[0, 0], twh, tbh, two[0],
                                tbo[0, 0], pre=pre3))
    hs = jnp.concatenate(_pe_t(uvt, 4) + [proc], axis=0)
    rect = _mlp_t(hs, swi[0], sbi[0, 0], swh, sbh, swo[0], sbo[0, 0],
                  pre=pre3)
    out_ref[:, 0:2] = jnp.transpose(uvt)
    out_ref[:, 2:5] = jnp.transpose(rect)


def _expert_call(eid, xs, proc, twi, tbi, twh, tbh, two, tbo,
                 swi, sbi, swh, sbh, swo, sbo):
    tbi, tbo, sbi, sbo = (a.reshape(C, 1, -1) for a in (tbi, tbo, sbi, sbo))
    e3 = lambda b, eid_ref: (eid_ref[b], 0, 0)
    e4 = lambda b, eid_ref: (eid_ref[b], 0, 0, 0)
    grid_spec = pltpu.PrefetchScalarGridSpec(
        num_scalar_prefetch=1,
        grid=(NB,),
        in_specs=[
            pl.BlockSpec((B, NPEW), lambda b, eid_ref: (b, 0)),
            pl.BlockSpec((1, 2), lambda b, eid_ref: (0, 0)),
            pl.BlockSpec((1, 29, H), e3),
            pl.BlockSpec((1, 1, H), e3),
            pl.BlockSpec((1, NHID, H, H), e4),
            pl.BlockSpec((1, NHID, H), e3),
            pl.BlockSpec((1, H, 2), e3),
            pl.BlockSpec((1, 1, 2), e3),
            pl.BlockSpec((1, 20, H), e3),
            pl.BlockSpec((1, 1, H), e3),
            pl.BlockSpec((1, NHID, H, H), e4),
            pl.BlockSpec((1, NHID, H), e3),
            pl.BlockSpec((1, H, 3), e3),
            pl.BlockSpec((1, 1, 3), e3),
        ],
        out_specs=pl.BlockSpec((B, NPEW), lambda b, eid_ref: (b, 0)),
    )
    return pl.pallas_call(
        _expert_body,
        grid_spec=grid_spec,
        out_shape=jax.ShapeDtypeStruct((NP, NPEW), jnp.float32),
    )(eid, xs, proc, twi, tbi, twh, tbh, two, tbo,
      swi, sbi, swh, sbh, swo, sbo)


def kernel(x, proc_params, cW_in, cb_in, cW_h, cb_h, cW_out, cb_out,
           tW_in, tb_in, tW_h, tb_h, tW_out, tb_out,
           sW_in, sb_in, sW_h, sb_h, sW_out, sb_out):
    probst, ps, eid = _router_call(x.T, proc_params, cW_in, cb_in,
                                   cW_h, cb_h, cW_out, cb_out)
    probs = probst.T
    x16 = jnp.concatenate(
        [x, jnp.zeros((N, NPEW - 3), jnp.float32)], axis=-1)
    ps2 = ps.reshape(N // 128, 128)
    xs = _sc_scatter_call(x16, ps2)
    ys = _expert_call(eid, xs, proc_params, tW_in, tb_in, tW_h, tb_h,
                      tW_out, tb_out, sW_in, sb_in, sW_h, sb_h,
                      sW_out, sb_out)
    g = _sc_gather_call(ys, ps2)
    return jnp.concatenate([probs, g[:, 0:5]], axis=-1)


# speedup vs baseline: 1.4516x; 1.2733x over previous
import functools

import jax
import jax.numpy as jnp
import numpy as np
from jax import lax
from jax.experimental import pallas as pl
from jax.experimental.pallas import tpu as pltpu
from jax.experimental.pallas import tpu_sc as plsc

N = 8192
C = 8
H = 256
NHID = 6
B = 512
NB = N // B + C
NP = NB * B
LCH = 4
NPEW = 128

_NC = 2
_NS = 16
_NW = _NC * _NS
_PW = N // _NW
_IC = _PW // 128


def _dgt(w, a):
    return lax.dot_general(w, a, (((0,), (0,)), ((), ())),
                           preferred_element_type=jnp.float32)


def _pe_t(v, degree):
    s = jnp.sin((2.0 * np.pi) * v)
    c = jnp.cos((2.0 * np.pi) * v)
    outs = [v, s, c]
    for _ in range(degree - 1):
        s, c = 2.0 * s * c, 1.0 - 2.0 * s * s
        outs.append(s)
        outs.append(c)
    return outs


def _mlp_t(h, w_in, b_in, w_h_ref, b_h_ref, w_out, b_out,
           pre=lambda a, b: a[b]):
    h = jax.nn.relu(_dgt(w_in, h) + b_in[:, None])
    for i in range(NHID):
        h = jax.nn.relu(_dgt(pre(w_h_ref, i), h) + pre(b_h_ref, i)[:, None])
    return _dgt(w_out, h) + b_out[:, None]


def _mlp(h, w_in, b_in, w_h_ref, b_h_ref, w_out, b_out,
         pre=lambda a, b: a[b]):
    h = jax.nn.relu(jnp.dot(h, w_in, preferred_element_type=jnp.float32)
                    + b_in)
    for i in range(NHID):
        h = jax.nn.relu(
            jnp.dot(h, pre(w_h_ref, i), preferred_element_type=jnp.float32)
            + pre(b_h_ref, i))
    return jnp.dot(h, w_out, preferred_element_type=jnp.float32) + b_out


def _router_body(xt_ref, proc_ref, wi_ref, bi_ref, wh_ref, bh_ref, wo_ref,
                 bo_ref, probst_ref, ps_ref, eid_ref):
    nn = N // LCH
    proc = jnp.broadcast_to(jnp.transpose(proc_ref[...]), (2, nn))
    wi = wi_ref[...]
    bi = bi_ref[...]
    wo = wo_ref[...]
    bo = bo_ref[...]
    for ch in range(LCH):
        sl = pl.ds(ch * nn, nn)
        xc = xt_ref[:, sl]
        pe1 = _pe_t(xc, 1)
        hc = jnp.concatenate(pe1 + [proc], axis=0)
        logits = _mlp_t(hc, wi, bi, wh_ref, bh_ref, wo, bo)
        m = jnp.max(logits, axis=0, keepdims=True)
        e = jnp.exp(logits - m)
        probst_ref[:, sl] = e / jnp.sum(e, axis=0, keepdims=True)
    p = probst_ref[...]
    pm = jnp.max(p, axis=0, keepdims=True)
    row = lax.broadcasted_iota(jnp.int32, (C, N), 0)
    idx = jnp.min(jnp.where(p == pm, row, C), axis=0, keepdims=True)
    onehot = (row == idx).astype(jnp.int32)
    cs = onehot
    s = 1
    while s < N:
        shifted = jnp.concatenate(
            [jnp.zeros((C, s), jnp.int32), cs[:, :N - s]], axis=1)
        cs = cs + shifted
        s *= 2
    rank = jnp.sum(cs * onehot, axis=0, keepdims=True) - 1
    counts = cs[:, N - 1:N]
    pg = ((counts + (B - 1)) // B) * B
    poff_incl = pg
    s = 1
    while s < C:
        poff_incl = poff_incl + jnp.concatenate(
            [jnp.zeros((s, 1), jnp.int32), poff_incl[:C - s, :]], axis=0)
        s *= 2
    poff_excl = poff_incl - pg
    ps_ref[...] = jnp.sum(onehot * poff_excl, axis=0, keepdims=True) + rank
    bstart = lax.broadcasted_iota(jnp.int32, (C, NB), 1) * B
    ge = (bstart >= poff_incl).astype(jnp.int32)
    eid_ref[...] = jnp.minimum(jnp.sum(ge, axis=0), C - 1)


def _router_call(xt, proc, wi, bi, wh, bh, wo, bo):
    return pl.pallas_call(
        _router_body,
        out_shape=(jax.ShapeDtypeStruct((C, N), jnp.float32),
                   jax.ShapeDtypeStruct((1, N), jnp.int32),
                   jax.ShapeDtypeStruct((NB,), jnp.int32)),
    )(xt, proc, wi, bi, wh, bh, wo, bo)


@functools.lru_cache(maxsize=None)
def _sc_kernels():
    mesh = plsc.VectorSubcoreMesh(core_axis_name="c", subcore_axis_name="s")
    scratch = [
        pltpu.VMEM((_IC, 128), jnp.int32),
        pltpu.VMEM((_PW, NPEW), jnp.float32),
        pltpu.SemaphoreType.DMA,
    ]

    @functools.partial(
        pl.kernel,
        out_type=jax.ShapeDtypeStruct((NP, NPEW), jnp.float32),
        mesh=mesh, scratch_types=scratch)
    def sc_scatter(x16_hbm, ps2_hbm, out_hbm, idx_v, rows_v, sem):
        wid = lax.axis_index("s") * _NC + lax.axis_index("c")
        base = wid * _PW
        pltpu.sync_copy(ps2_hbm.at[pl.ds(wid * _IC, _IC)], idx_v)
        pltpu.sync_copy(x16_hbm.at[pl.ds(base, _PW)], rows_v)
        for cidx in range(_IC):
            pltpu.async_copy(rows_v.at[pl.ds(cidx * 128, 128)],
                             out_hbm.at[idx_v.at[cidx]], sem).wait()

    @functools.partial(
        pl.kernel,
        out_type=jax.ShapeDtypeStruct((N, NPEW), jnp.float32),
        mesh=mesh, scratch_types=scratch)
    def sc_gather(ys_hbm, ps2_hbm, out_hbm, idx_v, rows_v, sem):
        wid = lax.axis_index("s") * _NC + lax.axis_index("c")
        base = wid * _PW
        pltpu.sync_copy(ps2_hbm.at[pl.ds(wid * _IC, _IC)], idx_v)
        for cidx in range(_IC):
            pltpu.async_copy(ys_hbm.at[idx_v.at[cidx]],
                             rows_v.at[pl.ds(cidx * 128, 128)], sem).wait()
        pltpu.sync_copy(rows_v, out_hbm.at[pl.ds(base, _PW)])

    return sc_scatter, sc_gather


def _sc_scatter_call(x16, ps2):
    return _sc_kernels()[0](x16, ps2)


def _sc_gather_call(ys, ps2):
    return _sc_kernels()[1](ys, ps2)


def _expert_body(eid_ref, xs_ref, proc_ref, twi, tbi, twh, tbh, two, tbo,
                 swi, sbi, swh, sbh, swo, sbo, out_ref):
    del eid_ref
    xbt = jnp.transpose(xs_ref[...][:, 0:3])
    proc = jnp.broadcast_to(jnp.transpose(proc_ref[...]), (2, B))
    ht = jnp.transpose(
        jnp.concatenate(_pe_t(xbt, 4) + [proc], axis=0))
    pre3 = lambda r, i: r[0, i]
    uv = jax.nn.sigmoid(_mlp(ht, twi[0], tbi[0, 0], twh, tbh, two[0],
                             tbo[0, 0], pre=pre3))
    hs = jnp.transpose(
        jnp.concatenate(_pe_t(jnp.transpose(uv), 4) + [proc],
                        axis=0))
    rec = _mlp(hs, swi[0], sbi[0, 0], swh, sbh, swo[0], sbo[0, 0],
               pre=pre3)
    out_ref[:, 0:2] = uv
    out_ref[:, 2:5] = rec


def _expert_call(eid, xs, proc, twi, tbi, twh, tbh, two, tbo,
                 swi, sbi, swh, sbh, swo, sbo):
    tbi, tbo, sbi, sbo = (a.reshape(C, 1, -1) for a in (tbi, tbo, sbi, sbo))
    e3 = lambda b, eid_ref: (eid_ref[b], 0, 0)
    e4 = lambda b, eid_ref: (eid_ref[b], 0, 0, 0)
    grid_spec = pltpu.PrefetchScalarGridSpec(
        num_scalar_prefetch=1,
        grid=(NB,),
        in_specs=[
            pl.BlockSpec((B, NPEW), lambda b, eid_ref: (b, 0)),
            pl.BlockSpec((1, 2), lambda b, eid_ref: (0, 0)),
            pl.BlockSpec((1, 29, H), e3),
            pl.BlockSpec((1, 1, H), e3),
            pl.BlockSpec((1, NHID, H, H), e4),
            pl.BlockSpec((1, NHID, H), e3),
            pl.BlockSpec((1, H, 2), e3),
            pl.BlockSpec((1, 1, 2), e3),
            pl.BlockSpec((1, 20, H), e3),
            pl.BlockSpec((1, 1, H), e3),
            pl.BlockSpec((1, NHID, H, H), e4),
            pl.BlockSpec((1, NHID, H), e3),
            pl.BlockSpec((1, H, 3), e3),
            pl.BlockSpec((1, 1, 3), e3),
        ],
        out_specs=pl.BlockSpec((B, NPEW), lambda b, eid_ref: (b, 0)),
    )
    return pl.pallas_call(
        _expert_body,
        grid_spec=grid_spec,
        out_shape=jax.ShapeDtypeStruct((NP, NPEW), jnp.float32),
    )(eid, xs, proc, twi, tbi, twh, tbh, two, tbo,
      swi, sbi, swh, sbh, swo, sbo)


def kernel(x, proc_params, cW_in, cb_in, cW_h, cb_h, cW_out, cb_out,
           tW_in, tb_in, tW_h, tb_h, tW_out, tb_out,
           sW_in, sb_in, sW_h, sb_h, sW_out, sb_out):
    probst, ps, eid = _router_call(x.T, proc_params, cW_in, cb_in,
                                   cW_h, cb_h, cW_out, cb_out)
    probs = probst.T
    x16 = jnp.concatenate(
        [x, jnp.zeros((N, NPEW - 3), jnp.float32)], axis=-1)
    ps2 = ps.reshape(N // 128, 128)
    xs = _sc_scatter_call(x16, ps2)
    ys = _expert_call(eid, xs, proc_params, tW_in, tb_in, tW_h, tb_h,
                      tW_out, tb_out, sW_in, sb_in, sW_h, sb_h,
                      sW_out, sb_out)
    g = _sc_gather_call(ys, ps2)
    return jnp.concatenate([probs, g[:, 0:5]], axis=-1)
